# baseline (speedup 1.0000x reference)
import functools

import jax
import jax.numpy as jnp
from jax import lax
from jax.experimental import pallas as pl
from jax.experimental.pallas import tpu as pltpu
from jax.experimental.pallas import tpu_sc as plsc

_NX, _NY, _NZ = 512, 512, 1
_C = 128
_P = 60000
_CELLS = _NZ * _NY * _NX
_NW = 32
_STRIPE = _CELLS // _NW
_SUB = 128
_NSUB = _STRIPE // _SUB
_CH1 = 4000
_NCH1 = _P // _CH1
_UNROLL1 = 5
_LISTCAP = _STRIPE + 16
_ROWCAP = _SUB + 16


def _iota16():
    return lax.iota(jnp.int32, 16)


def _body(idx_hbm, pf_hbm, out_hbm,
          w_map, ibuf, plist, dlist, rows, otile, segs,
          sem_row, sem_out0, sem_out1):
    wid = lax.axis_index("s") * 2 + lax.axis_index("c")
    base = wid * _STRIPE
    iota = _iota16()
    zerosf = jnp.zeros((16,), jnp.float32)

    def initw(i, _):
        w_map[pl.ds(i * 16, 16)] = jnp.full((16,), -1, jnp.int32)
        return 0
    lax.fori_loop(0, _STRIPE // 16, initw, 0)

    for b in range(2):
        def inito(c, _):
            for cg in range(_SUB // 16):
                otile[b, c, pl.ds(cg * 16, 16)] = zerosf
            return 0
        lax.fori_loop(0, _C, inito, 0)

    def p1_chunk(ci, _):
        off = ci * _CH1
        pltpu.sync_copy(idx_hbm.at[pl.ds(off, _CH1)], ibuf)

        def grp(g, _):
            os_ = [g * (16 * _UNROLL1) + u * 16 for u in range(_UNROLL1)]
            idxs = [ibuf[pl.ds(o, 16)] for o in os_]
            scans = [plsc.scan_count(ix)[1] for ix in idxs]
            for o, ix, lastm in zip(os_, idxs, scans):
                lidx = ix - base
                inr = (lidx >= 0) & (lidx < _STRIPE)
                m = lastm & inr
                plsc.store_scatter(w_map, [lidx], off + o + iota, mask=m)
            return 0
        lax.fori_loop(0, _CH1 // (16 * _UNROLL1), grp, 0)
        return 0
    lax.fori_loop(0, _NCH1, p1_chunk, 0)

    def comp(s, cur):
        segs[s] = cur

        def cgrp(j, cur2):
            w = w_map[pl.ds(s * _SUB + j * 16, 16)]
            m = w >= 0
            plsc.store_compressed(plist.at[pl.ds(cur2, 16)], w, mask=m)
            plsc.store_compressed(dlist.at[pl.ds(cur2, 16)],
                                  j * 16 + iota, mask=m)
            return cur2 + plsc.all_reduce_population_count(m)[0]
        return lax.fori_loop(0, _SUB // 16, cgrp, cur)
    ktot = lax.fori_loop(0, _NSUB, comp, jnp.int32(0))
    segs[_NSUB] = ktot
    plsc.store_compressed(plist.at[pl.ds(ktot, 16)],
                          jnp.zeros((16,), jnp.int32),
                          mask=jnp.full((16,), True))
    plsc.store_compressed(dlist.at[pl.ds(ktot, 16)],
                          jnp.zeros((16,), jnp.int32),
                          mask=jnp.full((16,), True))

    def half(s, b, pseg, sem_out):
        pseg0, pseg1 = pseg
        y0 = wid * 16 + (s >> 2)
        ty = y0 >> 3
        y8 = y0 & 7
        tx = s & 3

        seg0 = segs[s]
        seg1 = segs[s + 1]
        ga = seg0 >> 4
        gb = (seg1 + 15) >> 4

        def fire(g, _):
            pvec = plist[pl.ds(g * 16, 16)]
            pltpu.make_async_copy(
                pf_hbm.at[pvec],
                rows.at[pl.ds((g - ga) * 16, 16), :], sem_row).start()
            return 0
        lax.fori_loop(ga, gb, fire, 0)

        @pl.when(s >= 2)
        def _wait_prev():
            pltpu.make_async_copy(
                otile.at[b], out_hbm.at[:, ty, tx, y8, :], sem_out).wait()

        def rgrp(g, _):
            dvec = dlist[pl.ds(g * 16, 16)]
            for i in range(16):
                dloc = dvec[i]
                for cg in range(_C // 16):
                    plsc.store_scatter(
                        otile,
                        [jnp.full((16,), b, jnp.int32), cg * 16 + iota,
                         jnp.broadcast_to(dloc, (16,))],
                        zerosf)
            return 0
        lax.fori_loop(pseg0 >> 4, (pseg1 + 15) >> 4, rgrp, 0)

        def drain(g, _):
            pvec = plist[pl.ds(0, 16)]
            pltpu.make_async_copy(
                pf_hbm.at[pvec], rows.at[pl.ds(0, 16), :], sem_row).wait()
            return 0
        lax.fori_loop(ga, gb, drain, 0)

        def agrp(g, _):
            dvec = dlist[pl.ds(g * 16, 16)]

            def ldp(i):
                krow = (g - ga) * 16 + i
                return [rows[krow, pl.ds(cg * 16, 16)]
                        for cg in range(_C // 16)]
            vecs = ldp(0)
            for i in range(16):
                k = g * 16 + i
                m = jnp.broadcast_to((k >= seg0) & (k < seg1), (16,))
                dloc = dvec[i]
                nxt = ldp(i + 1) if i < 15 else None
                for cg in range(_C // 16):
                    plsc.store_scatter(
                        otile,
                        [jnp.full((16,), b, jnp.int32), cg * 16 + iota,
                         jnp.broadcast_to(dloc, (16,))],
                        vecs[cg], mask=m)
                vecs = nxt
            return 0
        lax.fori_loop(ga, gb, agrp, 0)

        pltpu.make_async_copy(
            otile.at[b], out_hbm.at[:, ty, tx, y8, :], sem_out).start()
        return (seg0, seg1)

    def pair(sp, carry):
        pa, pb = carry
        pa = half(sp * 2, 0, pa, sem_out0)
        pb = half(sp * 2 + 1, 1, pb, sem_out1)
        return (pa, pb)

    z = jnp.int32(0)
    lax.fori_loop(0, _NSUB // 2, pair, ((z, z), (z, z)))

    ty0 = wid * 2
    pltpu.make_async_copy(
        otile.at[0], out_hbm.at[:, ty0, 0, 0, :], sem_out0).wait()
    pltpu.make_async_copy(
        otile.at[1], out_hbm.at[:, ty0, 1, 0, :], sem_out1).wait()


_mesh = plsc.VectorSubcoreMesh(core_axis_name="c", subcore_axis_name="s")

_scatter = functools.partial(
    pl.kernel,
    out_type=jax.ShapeDtypeStruct((_C, _NY // 8, _NX // 128, 8, 128),
                                  jnp.float32),
    mesh=_mesh,
    compiler_params=pltpu.CompilerParams(use_tc_tiling_on_sc=False,
                                         needs_layout_passes=False),
    scratch_types=[
        pltpu.VMEM((_STRIPE,), jnp.int32),
        pltpu.VMEM((_CH1,), jnp.int32),
        pltpu.VMEM((_LISTCAP,), jnp.int32),
        pltpu.VMEM((_LISTCAP,), jnp.int32),
        pltpu.VMEM((_ROWCAP, _C), jnp.float32),
        pltpu.VMEM((2, _C, _SUB), jnp.float32),
        pltpu.SMEM((_NSUB + 2,), jnp.int32),
        pltpu.SemaphoreType.DMA,
        pltpu.SemaphoreType.DMA,
        pltpu.SemaphoreType.DMA,
    ],
)(_body)


def kernel(pillar_features, coords):
    ci = coords.astype(jnp.int32)
    idx = ci[:, 1] * (_NY * _NX) + ci[:, 2] * _NX + ci[:, 3]
    pf = pillar_features.astype(jnp.float32)
    out = _scatter(idx, pf)
    out = jnp.transpose(out, (0, 1, 3, 2, 4))
    return out.reshape(1, _C * _NZ, _NY, _NX)


# speedup vs baseline: 1.9751x; 1.9751x over previous
import functools

import jax
import jax.numpy as jnp
from jax import lax
from jax.experimental import pallas as pl
from jax.experimental.pallas import tpu as pltpu
from jax.experimental.pallas import tpu_sc as plsc

_NX, _NY, _NZ = 512, 512, 1
_C = 128
_P = 60000
_CELLS = _NZ * _NY * _NX
_NW = 32
_STRIPE = _CELLS // _NW
_SUB = 256
_NSUB = _STRIPE // _SUB
_CH1 = 2000
_NCH1 = _P // _CH1
_UNROLL1 = 5
_LISTCAP = _STRIPE + 16
_OPAD = 264
_HOFF = 136
_WAVE = 8
_ROWCAP = _WAVE * 16 + 16


def _iota16():
    return lax.iota(jnp.int32, 16)


def _body(idx_hbm, pf_hbm, out_hbm,
          w_map, ibuf, plist, dlist, rows, otile, segs,
          sem_row, sem_out0, sem_out1):
    wid = lax.axis_index("s") * 2 + lax.axis_index("c")
    base = wid * _STRIPE
    iota = _iota16()
    zerosf = jnp.zeros((16,), jnp.float32)

    def initw(i, _):
        w_map[pl.ds(i * 16, 16)] = jnp.full((16,), -1, jnp.int32)
        return 0
    lax.fori_loop(0, _STRIPE // 16, initw, 0)

    for b in range(2):
        def inito(c, _):
            for cg in range(_SUB // 16):
                otile[b, c, pl.ds(cg * 16, 16)] = zerosf
            otile[b, c, pl.ds(_OPAD - 16, 16)] = zerosf
            return 0
        lax.fori_loop(0, _C, inito, 0)

    def p1_chunk(ci, _):
        off = ci * _CH1
        pltpu.sync_copy(idx_hbm.at[pl.ds(off, _CH1)], ibuf)

        def grp(g, _):
            os_ = [g * (16 * _UNROLL1) + u * 16 for u in range(_UNROLL1)]
            idxs = [ibuf[pl.ds(o, 16)] for o in os_]
            scans = [plsc.scan_count(ix)[1] for ix in idxs]
            for o, ix, lastm in zip(os_, idxs, scans):
                lidx = ix - base
                inr = (lidx >= 0) & (lidx < _STRIPE)
                m = lastm & inr
                plsc.store_scatter(w_map, [lidx], off + o + iota, mask=m)
            return 0
        lax.fori_loop(0, _CH1 // (16 * _UNROLL1), grp, 0)
        return 0
    lax.fori_loop(0, _NCH1, p1_chunk, 0)

    def comp(s, cur):
        segs[s] = cur

        def cgrp(j, cur2):
            w = w_map[pl.ds(s * _SUB + j * 16, 16)]
            m = w >= 0
            plsc.store_compressed(plist.at[pl.ds(cur2, 16)], w, mask=m)
            plsc.store_compressed(dlist.at[pl.ds(cur2, 16)],
                                  j * 16 + iota, mask=m)
            return cur2 + plsc.all_reduce_population_count(m)[0]
        return lax.fori_loop(0, _SUB // 16, cgrp, cur)
    ktot = lax.fori_loop(0, _NSUB, comp, jnp.int32(0))
    segs[_NSUB] = ktot
    plsc.store_compressed(plist.at[pl.ds(ktot, 16)],
                          jnp.zeros((16,), jnp.int32),
                          mask=jnp.full((16,), True))
    plsc.store_compressed(dlist.at[pl.ds(ktot, 16)],
                          jnp.zeros((16,), jnp.int32),
                          mask=jnp.full((16,), True))

    def fire_wave(wa, wb, wbase):
        def fire(g, _):
            pvec = plist[pl.ds(g * 16, 16)]
            pltpu.make_async_copy(
                pf_hbm.at[pvec],
                rows.at[pl.ds((g - wbase) * 16, 16), :], sem_row).start()
            return 0
        lax.fori_loop(wa, wb, fire, 0)

    def drain_wave(wa, wb):
        def drain(g, _):
            pvec = plist[pl.ds(0, 16)]
            pltpu.make_async_copy(
                pf_hbm.at[pvec], rows.at[pl.ds(0, 16), :], sem_row).wait()
            return 0
        lax.fori_loop(wa, wb, drain, 0)

    def assemble_wave(b, seg0, seg1, wa, wb, wbase):
        def agrp(g, _):
            dvec = dlist[pl.ds(g * 16, 16)]

            def ldp(i):
                krow = (g - wbase) * 16 + i
                return [rows[krow, pl.ds(cg * 16, 16)]
                        for cg in range(_C // 16)]
            vecs = ldp(0)
            for i in range(16):
                k = g * 16 + i
                m = jnp.broadcast_to((k >= seg0) & (k < seg1), (16,))
                scol = dvec[i] + ((dvec[i] & 128) >> 4)
                nxt = ldp(i + 1) if i < 15 else None
                for cg in range(_C // 16):
                    plsc.store_scatter(
                        otile,
                        [jnp.full((16,), b, jnp.int32), cg * 16 + iota,
                         jnp.broadcast_to(scol, (16,))],
                        vecs[cg], mask=m)
                vecs = nxt
            return 0
        lax.fori_loop(wa, wb, agrp, 0)

    def half(s, b, pseg, sem_out):
        pseg0, pseg1 = pseg
        y0 = wid * 16 + (s >> 1)
        ty = y0 >> 3
        y8 = y0 & 7

        seg0 = segs[s]
        seg1 = segs[s + 1]
        ga = seg0 >> 4
        gb = (seg1 + 15) >> 4
        gmid = jnp.minimum(ga + _WAVE, gb)

        fire_wave(ga, gmid, ga)

        @pl.when(s >= 2)
        def _wait_prev():
            for j in range(2):
                pltpu.make_async_copy(
                    otile.at[b, :, pl.ds(j * _HOFF, 128)],
                    out_hbm.at[:, ty, b * 2 + j, y8, :], sem_out).wait()

        def rgrp(g, _):
            dvec = dlist[pl.ds(g * 16, 16)]
            for i in range(16):
                scol = dvec[i] + ((dvec[i] & 128) >> 4)
                for cg in range(_C // 16):
                    plsc.store_scatter(
                        otile,
                        [jnp.full((16,), b, jnp.int32), cg * 16 + iota,
                         jnp.broadcast_to(scol, (16,))],
                        zerosf)
            return 0
        lax.fori_loop(pseg0 >> 4, (pseg1 + 15) >> 4, rgrp, 0)

        drain_wave(ga, gmid)
        assemble_wave(b, seg0, seg1, ga, gmid, ga)

        @pl.when(gmid < gb)
        def _wave2():
            fire_wave(gmid, gb, gmid)
            drain_wave(gmid, gb)
            assemble_wave(b, seg0, seg1, gmid, gb, gmid)

        for j in range(2):
            pltpu.make_async_copy(
                otile.at[b, :, pl.ds(j * _HOFF, 128)],
                out_hbm.at[:, ty, b * 2 + j, y8, :], sem_out).start()
        return (seg0, seg1)

    def pair(sp, carry):
        pa, pb = carry
        pa = half(sp * 2, 0, pa, sem_out0)
        pb = half(sp * 2 + 1, 1, pb, sem_out1)
        return (pa, pb)

    z = jnp.int32(0)
    lax.fori_loop(0, _NSUB // 2, pair, ((z, z), (z, z)))

    ty0 = wid * 2
    for j in range(2):
        pltpu.make_async_copy(
            otile.at[0, :, pl.ds(j * _HOFF, 128)],
            out_hbm.at[:, ty0, j, 0, :], sem_out0).wait()
    for j in range(2):
        pltpu.make_async_copy(
            otile.at[1, :, pl.ds(j * _HOFF, 128)],
            out_hbm.at[:, ty0, 2 + j, 0, :], sem_out1).wait()


_mesh = plsc.VectorSubcoreMesh(core_axis_name="c", subcore_axis_name="s")

_scatter = functools.partial(
    pl.kernel,
    out_type=jax.ShapeDtypeStruct((_C, _NY // 8, _NX // 128, 8, 128),
                                  jnp.float32),
    mesh=_mesh,
    compiler_params=pltpu.CompilerParams(use_tc_tiling_on_sc=False,
                                         needs_layout_passes=False),
    scratch_types=[
        pltpu.VMEM((_STRIPE,), jnp.int32),
        pltpu.VMEM((_CH1,), jnp.int32),
        pltpu.VMEM((_LISTCAP,), jnp.int32),
        pltpu.VMEM((_LISTCAP,), jnp.int32),
        pltpu.VMEM((_ROWCAP, _C), jnp.float32),
        pltpu.VMEM((2, _C, _OPAD), jnp.float32),
        pltpu.SMEM((_NSUB + 2,), jnp.int32),
        pltpu.SemaphoreType.DMA,
        pltpu.SemaphoreType.DMA,
        pltpu.SemaphoreType.DMA,
    ],
)(_body)


def kernel(pillar_features, coords):
    ci = coords.astype(jnp.int32)
    idx = ci[:, 1] * (_NY * _NX) + ci[:, 2] * _NX + ci[:, 3]
    pf = pillar_features.astype(jnp.float32)
    out = _scatter(idx, pf)
    out = jnp.transpose(out, (0, 1, 3, 2, 4))
    return out.reshape(1, _C * _NZ, _NY, _NX)


# speedup vs baseline: 1.9813x; 1.0031x over previous
import functools

import jax
import jax.numpy as jnp
from jax import lax
from jax.experimental import pallas as pl
from jax.experimental.pallas import tpu as pltpu
from jax.experimental.pallas import tpu_sc as plsc

_NX, _NY, _NZ = 512, 512, 1
_C = 128
_P = 60000
_CELLS = _NZ * _NY * _NX
_NW = 32
_STRIPE = _CELLS // _NW
_SUB = 256
_NSUB = _STRIPE // _SUB
_CH1 = 2000
_NCH1 = _P // _CH1
_UNROLL1 = 5
_LISTCAP = _STRIPE + 16
_OPAD = 264
_HOFF = 136
_WAVE = 8
_ROWCAP = _WAVE * 16 + 16


def _iota16():
    return lax.iota(jnp.int32, 16)


def _body(idx_hbm, pf_hbm, out_hbm,
          w_map, ibuf, plist, dlist, rows, otile, segs,
          sem_row, sem_out0, sem_out1):
    wid = lax.axis_index("s") * 2 + lax.axis_index("c")
    base = wid * _STRIPE
    iota = _iota16()
    zerosf = jnp.zeros((16,), jnp.float32)

    _s0 = jax.named_scope("ph0_init"); _s0.__enter__()
    def initw(i, _):
        w_map[pl.ds(i * 16, 16)] = jnp.full((16,), -1, jnp.int32)
        return 0
    lax.fori_loop(0, _STRIPE // 16, initw, 0)

    for b in range(2):
        def inito(c, _):
            for cg in range(_SUB // 16):
                otile[b, c, pl.ds(cg * 16, 16)] = zerosf
            otile[b, c, pl.ds(_OPAD - 16, 16)] = zerosf
            return 0
        lax.fori_loop(0, _C, inito, 0)

    _s0.__exit__(None, None, None)
    _s1 = jax.named_scope("ph1_winner"); _s1.__enter__()
    def p1_chunk(ci, _):
        off = ci * _CH1
        pltpu.sync_copy(idx_hbm.at[pl.ds(off, _CH1)], ibuf)

        def grp(g, _):
            os_ = [g * (16 * _UNROLL1) + u * 16 for u in range(_UNROLL1)]
            idxs = [ibuf[pl.ds(o, 16)] for o in os_]
            scans = [plsc.scan_count(ix)[1] for ix in idxs]
            for o, ix, lastm in zip(os_, idxs, scans):
                lidx = ix - base
                inr = (lidx >= 0) & (lidx < _STRIPE)
                m = lastm & inr
                plsc.store_scatter(w_map, [lidx], off + o + iota, mask=m)
            return 0
        lax.fori_loop(0, _CH1 // (16 * _UNROLL1), grp, 0)
        return 0
    lax.fori_loop(0, _NCH1, p1_chunk, 0)

    _s1.__exit__(None, None, None)
    _s2 = jax.named_scope("ph2_compact"); _s2.__enter__()
    def comp(s, cur):
        segs[s] = cur

        def cgrp(j, cur2):
            w = w_map[pl.ds(s * _SUB + j * 16, 16)]
            m = w >= 0
            plsc.store_compressed(plist.at[pl.ds(cur2, 16)], w, mask=m)
            plsc.store_compressed(dlist.at[pl.ds(cur2, 16)],
                                  j * 16 + iota, mask=m)
            return cur2 + plsc.all_reduce_population_count(m)[0]
        return lax.fori_loop(0, _SUB // 16, cgrp, cur)
    ktot = lax.fori_loop(0, _NSUB, comp, jnp.int32(0))
    segs[_NSUB] = ktot
    plsc.store_compressed(plist.at[pl.ds(ktot, 16)],
                          jnp.zeros((16,), jnp.int32),
                          mask=jnp.full((16,), True))
    plsc.store_compressed(dlist.at[pl.ds(ktot, 16)],
                          jnp.zeros((16,), jnp.int32),
                          mask=jnp.full((16,), True))

    _s2.__exit__(None, None, None)
    _s3 = jax.named_scope("ph3_assemble"); _s3.__enter__()
    def fire_wave(wa, wb, wbase):
        def fire(g, _):
            pvec = plist[pl.ds(g * 16, 16)]
            pltpu.make_async_copy(
                pf_hbm.at[pvec],
                rows.at[pl.ds((g - wbase) * 16, 16), :], sem_row).start()
            return 0
        lax.fori_loop(wa, wb, fire, 0)

    def drain_wave(wa, wb):
        def drain(g, _):
            pvec = plist[pl.ds(0, 16)]
            pltpu.make_async_copy(
                pf_hbm.at[pvec], rows.at[pl.ds(0, 16), :], sem_row).wait()
            return 0
        lax.fori_loop(wa, wb, drain, 0)

    def assemble_wave(b, seg0, seg1, wa, wb, wbase):
        def agrp(g, _):
            dvec = dlist[pl.ds(g * 16, 16)]

            def ldp(i):
                krow = (g - wbase) * 16 + i
                return [rows[krow, pl.ds(cg * 16, 16)]
                        for cg in range(_C // 16)]
            vecs = ldp(0)
            for i in range(16):
                k = g * 16 + i
                m = jnp.broadcast_to((k >= seg0) & (k < seg1), (16,))
                scol = dvec[i] + ((dvec[i] & 128) >> 4)
                nxt = ldp(i + 1) if i < 15 else None
                for cg in range(_C // 16):
                    plsc.store_scatter(
                        otile,
                        [jnp.full((16,), b, jnp.int32), cg * 16 + iota,
                         jnp.broadcast_to(scol, (16,))],
                        vecs[cg], mask=m)
                vecs = nxt
            return 0
        lax.fori_loop(wa, wb, agrp, 0)

    def half(s, b, pseg, sem_out):
        pseg0, pseg1 = pseg
        y0 = wid * 16 + (s >> 1)
        ty = y0 >> 3
        y8 = y0 & 7

        seg0 = segs[s]
        seg1 = segs[s + 1]
        ga = seg0 >> 4
        gb = (seg1 + 15) >> 4
        gmid = jnp.minimum(ga + _WAVE, gb)

        fire_wave(ga, gmid, ga)

        @pl.when(s >= 2)
        def _wait_prev():
            for j in range(2):
                pltpu.make_async_copy(
                    otile.at[b, :, pl.ds(j * _HOFF, 128)],
                    out_hbm.at[:, ty, b * 2 + j, y8, :], sem_out).wait()

        def rgrp(g, _):
            dvec = dlist[pl.ds(g * 16, 16)]
            for i in range(16):
                scol = dvec[i] + ((dvec[i] & 128) >> 4)
                for cg in range(_C // 16):
                    plsc.store_scatter(
                        otile,
                        [jnp.full((16,), b, jnp.int32), cg * 16 + iota,
                         jnp.broadcast_to(scol, (16,))],
                        zerosf)
            return 0
        lax.fori_loop(pseg0 >> 4, (pseg1 + 15) >> 4, rgrp, 0)

        drain_wave(ga, gmid)
        assemble_wave(b, seg0, seg1, ga, gmid, ga)

        @pl.when(gmid < gb)
        def _wave2():
            fire_wave(gmid, gb, gmid)
            drain_wave(gmid, gb)
            assemble_wave(b, seg0, seg1, gmid, gb, gmid)

        for j in range(2):
            pltpu.make_async_copy(
                otile.at[b, :, pl.ds(j * _HOFF, 128)],
                out_hbm.at[:, ty, b * 2 + j, y8, :], sem_out).start()
        return (seg0, seg1)

    def pair(sp, carry):
        pa, pb = carry
        pa = half(sp * 2, 0, pa, sem_out0)
        pb = half(sp * 2 + 1, 1, pb, sem_out1)
        return (pa, pb)

    z = jnp.int32(0)
    lax.fori_loop(0, _NSUB // 2, pair, ((z, z), (z, z)))

    _s3.__exit__(None, None, None)
    ty0 = wid * 2
    for j in range(2):
        pltpu.make_async_copy(
            otile.at[0, :, pl.ds(j * _HOFF, 128)],
            out_hbm.at[:, ty0, j, 0, :], sem_out0).wait()
    for j in range(2):
        pltpu.make_async_copy(
            otile.at[1, :, pl.ds(j * _HOFF, 128)],
            out_hbm.at[:, ty0, 2 + j, 0, :], sem_out1).wait()


_mesh = plsc.VectorSubcoreMesh(core_axis_name="c", subcore_axis_name="s")

_scatter = functools.partial(
    pl.kernel,
    out_type=jax.ShapeDtypeStruct((_C, _NY // 8, _NX // 128, 8, 128),
                                  jnp.float32),
    mesh=_mesh,
    compiler_params=pltpu.CompilerParams(use_tc_tiling_on_sc=False,
                                         needs_layout_passes=False),
    scratch_types=[
        pltpu.VMEM((_STRIPE,), jnp.int32),
        pltpu.VMEM((_CH1,), jnp.int32),
        pltpu.VMEM((_LISTCAP,), jnp.int32),
        pltpu.VMEM((_LISTCAP,), jnp.int32),
        pltpu.VMEM((_ROWCAP, _C), jnp.float32),
        pltpu.VMEM((2, _C, _OPAD), jnp.float32),
        pltpu.SMEM((_NSUB + 2,), jnp.int32),
        pltpu.SemaphoreType.DMA,
        pltpu.SemaphoreType.DMA,
        pltpu.SemaphoreType.DMA,
    ],
)(_body)


def kernel(pillar_features, coords):
    ci = coords.astype(jnp.int32)
    idx = ci[:, 1] * (_NY * _NX) + ci[:, 2] * _NX + ci[:, 3]
    pf = pillar_features.astype(jnp.float32)
    out = _scatter(idx, pf)
    out = jnp.transpose(out, (0, 1, 3, 2, 4))
    return out.reshape(1, _C * _NZ, _NY, _NX)


# speedup vs baseline: 2.2559x; 1.1386x over previous
import functools

import jax
import jax.numpy as jnp
from jax import lax
from jax.experimental import pallas as pl
from jax.experimental.pallas import tpu as pltpu
from jax.experimental.pallas import tpu_sc as plsc

_NX, _NY, _NZ = 512, 512, 1
_C = 128
_P = 60000
_CELLS = _NZ * _NY * _NX
_NW = 32
_STRIPE = _CELLS // _NW
_SUB = 256
_NSUB = _STRIPE // _SUB
_CH1 = 2000
_NCH1 = _P // _CH1
_UNROLL1 = 5
_LISTCAP = _STRIPE + 16
_OPAD = 264
_HOFF = 136
_WAVE = 6
_ROWCAP = _WAVE * 16 + 16


def _iota16():
    return lax.iota(jnp.int32, 16)


def _body(idx_hbm, pf_hbm, out_hbm,
          w_map, ibuf, plist, dlist, rows, otile, segs,
          sem_row0, sem_row1, sem_out0, sem_out1):
    wid = lax.axis_index("s") * 2 + lax.axis_index("c")
    base = wid * _STRIPE
    iota = _iota16()
    zerosf = jnp.zeros((16,), jnp.float32)

    def initw(i, _):
        w_map[pl.ds(i * 16, 16)] = jnp.full((16,), -1, jnp.int32)
        return 0
    lax.fori_loop(0, _STRIPE // 16, initw, 0)

    for b in range(2):
        def inito(c, _):
            for cg in range(_SUB // 16):
                otile[b, c, pl.ds(cg * 16, 16)] = zerosf
            otile[b, c, pl.ds(_OPAD - 16, 16)] = zerosf
            return 0
        lax.fori_loop(0, _C, inito, 0)

    p1_sems = (sem_row0, sem_row1)

    def p1_copy(ci, buf):
        pltpu.make_async_copy(
            idx_hbm.at[pl.ds(ci * _CH1, _CH1)], ibuf.at[buf],
            p1_sems[buf]).start()

    def p1_process(ci, buf):
        off = ci * _CH1
        pltpu.make_async_copy(
            idx_hbm.at[pl.ds(off, _CH1)], ibuf.at[buf], p1_sems[buf]).wait()

        def grp(g, _):
            os_ = [g * (16 * _UNROLL1) + u * 16 for u in range(_UNROLL1)]
            idxs = [ibuf[buf, pl.ds(o, 16)] for o in os_]
            scans = [plsc.scan_count(ix)[1] for ix in idxs]
            for o, ix, lastm in zip(os_, idxs, scans):
                lidx = ix - base
                inr = (lidx >= 0) & (lidx < _STRIPE)
                m = lastm & inr
                plsc.store_scatter(w_map, [lidx], off + o + iota, mask=m)
            return 0
        lax.fori_loop(0, _CH1 // (16 * _UNROLL1), grp, 0)

    p1_copy(jnp.int32(0), 0)

    def p1_pair(cp, _):
        c0 = cp * 2
        p1_copy(c0 + 1, 1)
        p1_process(c0, 0)

        @pl.when(cp < _NCH1 // 2 - 1)
        def _():
            p1_copy(c0 + 2, 0)
        p1_process(c0 + 1, 1)
        return 0
    lax.fori_loop(0, _NCH1 // 2, p1_pair, 0)

    def comp(s, cur):
        segs[s] = cur

        def cgrp(j, cur2):
            w = w_map[pl.ds(s * _SUB + j * 16, 16)]
            m = w >= 0
            plsc.store_compressed(plist.at[pl.ds(cur2, 16)], w, mask=m)
            plsc.store_compressed(dlist.at[pl.ds(cur2, 16)],
                                  j * 16 + iota, mask=m)
            return cur2 + plsc.all_reduce_population_count(m)[0]
        return lax.fori_loop(0, _SUB // 16, cgrp, cur)
    ktot = lax.fori_loop(0, _NSUB, comp, jnp.int32(0))
    segs[_NSUB] = ktot
    segs[_NSUB + 1] = ktot
    plsc.store_compressed(plist.at[pl.ds(ktot, 16)],
                          jnp.zeros((16,), jnp.int32),
                          mask=jnp.full((16,), True))
    plsc.store_compressed(dlist.at[pl.ds(ktot, 16)],
                          jnp.zeros((16,), jnp.int32),
                          mask=jnp.full((16,), True))

    sem_rows = (sem_row0, sem_row1)
    sem_outs = (sem_out0, sem_out1)

    def fire_wave(buf, wa, wb, wbase):
        def fire(g, _):
            pvec = plist[pl.ds(g * 16, 16)]
            pltpu.make_async_copy(
                pf_hbm.at[pvec],
                rows.at[buf, pl.ds((g - wbase) * 16, 16), :],
                sem_rows[buf]).start()
            return 0
        lax.fori_loop(wa, wb, fire, 0)

    def drain_wave(buf, wa, wb):
        def drain(g, _):
            pvec = plist[pl.ds(0, 16)]
            pltpu.make_async_copy(
                pf_hbm.at[pvec], rows.at[buf, pl.ds(0, 16), :],
                sem_rows[buf]).wait()
            return 0
        lax.fori_loop(wa, wb, drain, 0)

    def assemble_wave(buf, b, seg0, seg1, wa, wb, wbase):
        def agrp(g, _):
            dvec = dlist[pl.ds(g * 16, 16)]

            def ldp(i):
                krow = (g - wbase) * 16 + i
                return [rows[buf, krow, pl.ds(cg * 16, 16)]
                        for cg in range(_C // 16)]
            vecs = ldp(0)
            for i in range(16):
                k = g * 16 + i
                m = jnp.broadcast_to((k >= seg0) & (k < seg1), (16,))
                scol = dvec[i] + ((dvec[i] & 128) >> 4)
                nxt = ldp(i + 1) if i < 15 else None
                for cg in range(_C // 16):
                    plsc.store_scatter(
                        otile,
                        [jnp.full((16,), b, jnp.int32), cg * 16 + iota,
                         jnp.broadcast_to(scol, (16,))],
                        vecs[cg], mask=m)
                vecs = nxt
            return 0
        lax.fori_loop(wa, wb, agrp, 0)

    def wave1_bounds(s):
        seg0 = segs[s]
        seg1 = segs[s + 1]
        ga = seg0 >> 4
        gb = (seg1 + 15) >> 4
        return seg0, seg1, ga, jnp.minimum(ga + _WAVE, gb), gb

    def half(s, b, pseg, sem_out):
        pseg0, pseg1 = pseg
        y0 = wid * 16 + (s >> 1)
        ty = y0 >> 3
        y8 = y0 & 7

        seg0, seg1, ga, gmid, gb = wave1_bounds(s)

        @pl.when(s >= 2)
        def _wait_prev():
            for j in range(2):
                pltpu.make_async_copy(
                    otile.at[b, :, pl.ds(j * _HOFF, 128)],
                    out_hbm.at[:, ty, b * 2 + j, y8, :], sem_out).wait()

        def rgrp(g, _):
            dvec = dlist[pl.ds(g * 16, 16)]
            for i in range(16):
                scol = dvec[i] + ((dvec[i] & 128) >> 4)
                for cg in range(_C // 16):
                    plsc.store_scatter(
                        otile,
                        [jnp.full((16,), b, jnp.int32), cg * 16 + iota,
                         jnp.broadcast_to(scol, (16,))],
                        zerosf)
            return 0
        lax.fori_loop(pseg0 >> 4, (pseg1 + 15) >> 4, rgrp, 0)

        @pl.when(s < _NSUB - 1)
        def _prefetch():
            _, _, ga_n, gmid_n, _ = wave1_bounds(s + 1)
            fire_wave(1 - b, ga_n, gmid_n, ga_n)

        drain_wave(b, ga, gmid)
        assemble_wave(b, b, seg0, seg1, ga, gmid, ga)

        def wave2(w, _):
            wa = gmid + w * _WAVE
            wb = jnp.minimum(wa + _WAVE, gb)
            fire_wave(b, wa, wb, wa)
            drain_wave(b, wa, wb)
            assemble_wave(b, b, seg0, seg1, wa, wb, wa)
            return 0
        lax.fori_loop(0, (gb - gmid + _WAVE - 1) // _WAVE, wave2, 0)

        for j in range(2):
            pltpu.make_async_copy(
                otile.at[b, :, pl.ds(j * _HOFF, 128)],
                out_hbm.at[:, ty, b * 2 + j, y8, :], sem_out).start()
        return (seg0, seg1)

    _, _, ga0, gmid0, _ = wave1_bounds(0)
    fire_wave(0, ga0, gmid0, ga0)

    def pair(sp, carry):
        pa, pb = carry
        pa = half(sp * 2, 0, pa, sem_outs[0])
        pb = half(sp * 2 + 1, 1, pb, sem_outs[1])
        return (pa, pb)

    z = jnp.int32(0)
    lax.fori_loop(0, _NSUB // 2, pair, ((z, z), (z, z)))

    ty0 = wid * 2
    for j in range(2):
        pltpu.make_async_copy(
            otile.at[0, :, pl.ds(j * _HOFF, 128)],
            out_hbm.at[:, ty0, j, 0, :], sem_out0).wait()
    for j in range(2):
        pltpu.make_async_copy(
            otile.at[1, :, pl.ds(j * _HOFF, 128)],
            out_hbm.at[:, ty0, 2 + j, 0, :], sem_out1).wait()


_mesh = plsc.VectorSubcoreMesh(core_axis_name="c", subcore_axis_name="s")

_scatter = functools.partial(
    pl.kernel,
    out_type=jax.ShapeDtypeStruct((_C, _NY // 8, _NX // 128, 8, 128),
                                  jnp.float32),
    mesh=_mesh,
    compiler_params=pltpu.CompilerParams(use_tc_tiling_on_sc=False,
                                         needs_layout_passes=False),
    scratch_types=[
        pltpu.VMEM((_STRIPE,), jnp.int32),
        pltpu.VMEM((2, _CH1), jnp.int32),
        pltpu.VMEM((_LISTCAP,), jnp.int32),
        pltpu.VMEM((_LISTCAP,), jnp.int32),
        pltpu.VMEM((2, _ROWCAP, _C), jnp.float32),
        pltpu.VMEM((2, _C, _OPAD), jnp.float32),
        pltpu.SMEM((_NSUB + 2,), jnp.int32),
        pltpu.SemaphoreType.DMA,
        pltpu.SemaphoreType.DMA,
        pltpu.SemaphoreType.DMA,
        pltpu.SemaphoreType.DMA,
    ],
)(_body)


def kernel(pillar_features, coords):
    ci = coords.astype(jnp.int32)
    idx = ci[:, 1] * (_NY * _NX) + ci[:, 2] * _NX + ci[:, 3]
    pf = pillar_features.astype(jnp.float32)
    out = _scatter(idx, pf)
    out = jnp.transpose(out, (0, 1, 3, 2, 4))
    return out.reshape(1, _C * _NZ, _NY, _NX)
